# baseline (speedup 1.0000x reference)
import functools

import jax
import jax.numpy as jnp
from jax import lax
from jax.experimental import pallas as pl
from jax.experimental.pallas import tpu as pltpu
from jax.experimental.pallas import tpu_sc as plsc

_NC, _NS = 2, 16
_NW = _NC * _NS
_CHUNK = 80
_NBUF = 5


@functools.lru_cache(maxsize=None)
def _build(n_chunks, S, D):
    assert (_CHUNK * _NBUF) % S == 0
    assert n_chunks % (_NW * _NBUF) == 0
    ncw = n_chunks // _NW
    ngroups = ncw // _NBUF
    n_rows = n_chunks * _CHUNK
    mesh = plsc.VectorSubcoreMesh(core_axis_name="c", subcore_axis_name="s")

    @functools.partial(
        pl.kernel,
        out_type=jax.ShapeDtypeStruct((n_rows, D), jnp.float32),
        mesh=mesh,
        scratch_types=[
            pltpu.VMEM((ncw, _CHUNK), jnp.int32),
            pltpu.VMEM((2 * S, D), jnp.float32),
            pltpu.VMEM((_NBUF, _CHUNK, D), jnp.float32),
        ] + [pltpu.SemaphoreType.DMA] * (2 * _NBUF),
        compiler_params=pltpu.CompilerParams(use_tc_tiling_on_sc=False),
    )
    def k(x_hbm, table_hbm, pe_hbm, out_hbm, idx_v, pe_v, rows_v, *sems):
        gsems = sems[:_NBUF]
        ssems = sems[_NBUF:]
        wid = lax.axis_index("s") * _NC + lax.axis_index("c")
        c0 = wid * ncw
        pltpu.sync_copy(x_hbm.at[pl.ds(c0, ncw)], idx_v)
        pltpu.sync_copy(pe_hbm, pe_v)

        def fire_gather(c_local, b):
            pltpu.make_async_copy(
                table_hbm.at[idx_v.at[c_local]], rows_v.at[b], gsems[b]
            ).start()

        def wait_gather(b):
            pltpu.make_async_copy(
                table_hbm.at[idx_v.at[0]], rows_v.at[b], gsems[b]
            ).wait()

        def fire_scatter(c_local, b):
            row0 = (c0 + c_local) * _CHUNK
            pltpu.make_async_copy(
                rows_v.at[b], out_hbm.at[pl.ds(row0, _CHUNK)], ssems[b]
            ).start()

        def wait_scatter(b):
            pltpu.make_async_copy(
                rows_v.at[b], out_hbm.at[pl.ds(0, _CHUNK)], ssems[b]
            ).wait()

        def add_pe(b):
            q = (_CHUNK * b) % S

            def row_body(r, carry):
                for gi in range(D // 16):
                    sl = pl.ds(gi * 16, 16)
                    rows_v[b, r, sl] = rows_v[b, r, sl] + pe_v[q + r, sl]
                return carry

            lax.fori_loop(0, _CHUNK, row_body, 0, unroll=2)

        def chunk_step(c_local, b, first, last):
            wait_gather(b)
            add_pe(b)
            nb = (b + _NBUF - 1) % _NBUF
            if not last:
                if not first:
                    wait_scatter(nb)
                fire_gather(c_local + (_NBUF - 1), nb)
            fire_scatter(c_local, b)

        for b in range(_NBUF - 1):
            fire_gather(b, b)
        for b in range(_NBUF):
            chunk_step(b, b, first=(b == 0), last=False)

        def group_body(g, carry):
            cbase = g * _NBUF
            for b in range(_NBUF):
                chunk_step(cbase + b, b, first=False, last=False)
            return carry

        lax.fori_loop(1, ngroups - 1, group_body, 0)

        cbase = (ngroups - 1) * _NBUF
        for b in range(_NBUF):
            chunk_step(cbase + b, b, first=False, last=(b != 0))
        for b in range(_NBUF):
            wait_scatter(b)

    return k


def kernel(x, embedding_matrix, positional_encodings):
    B, S = x.shape
    V, D = embedding_matrix.shape
    n_chunks = B * S // _CHUNK
    x2 = x.reshape(n_chunks, _CHUNK).astype(jnp.int32)
    pe = positional_encodings[:S]
    pe2 = jnp.concatenate([pe, pe], axis=0)
    out = _build(n_chunks, S, D)(x2, embedding_matrix, pe2)
    return out.reshape(B, S, D)
